# baseline (speedup 1.0000x reference)
import functools

import jax
import jax.numpy as jnp
from jax import lax
from jax.experimental import pallas as pl
from jax.experimental.pallas import tpu as pltpu
from jax.experimental.pallas import tpu_sc as plsc

_T, _D, _H, _E, _O = 2048, 1024, 1024, 8, 1024
_K = 2
_BB = 512
_NPAD = 8192
_G = _NPAD // _BB
_NW = 32
_TPW = _T // _NW


def _route_body(x_ref, gw_ref, gb_ref, gow_ref, gob_ref,
                p016_ref, p116_ref, g016_ref, g116_ref,
                bexp_ref, brun_ref, nblk_ref, log_scr):
    i = pl.program_id(0)
    nb = pl.num_programs(0)
    bt = _T // nb
    h = jnp.dot(x_ref[...], gw_ref[...], preferred_element_type=jnp.float32)
    h = jnp.maximum(h + gb_ref[...], 0.0)
    log_scr[pl.ds(i * bt, bt), :] = (
        jnp.dot(h, gow_ref[...], preferred_element_type=jnp.float32)
        + gob_ref[...])

    @pl.when(i == nb - 1)
    def _():
        logits = log_scr[...]
        lane = lax.broadcasted_iota(jnp.int32, (_T, _E), 1)
        m1 = jnp.max(logits, axis=1, keepdims=True)
        i1 = jnp.min(jnp.where(logits == m1, lane, _E), axis=1, keepdims=True)
        masked = jnp.where(lane == i1, -jnp.inf, logits)
        m2 = jnp.max(masked, axis=1, keepdims=True)
        i2 = jnp.min(jnp.where(masked == m2, lane, _E), axis=1, keepdims=True)
        e2 = jnp.exp(m2 - m1)
        den = 1.0 + e2
        g1 = 1.0 / den
        g2 = e2 / den
        sel1 = jnp.where(lane == i1, 1.0, 0.0)
        sel2 = jnp.where(lane == i2, 1.0, 0.0)

        csum = sel1 + sel2
        sh = 1
        while sh < _T:
            csum = csum + jnp.concatenate(
                [jnp.zeros((sh, _E), jnp.float32), csum[:-sh, :]], axis=0)
            sh *= 2
        counts = csum[_T - 1:_T, :]
        pc = jnp.floor((counts + (_BB - 1)) * (1.0 / _BB)) * _BB
        incl = pc
        for sh2 in (1, 2, 4):
            incl = incl + jnp.concatenate(
                [jnp.zeros((1, sh2), jnp.float32), incl[:, :-sh2]], axis=1)
        po = incl - pc
        posb = csum + po - 1.0
        p0 = jnp.sum(sel1 * posb, axis=1, keepdims=True).astype(jnp.int32)
        p1 = jnp.sum(sel2 * posb, axis=1, keepdims=True).astype(jnp.int32)
        p016_ref[...] = jnp.reshape(p0, (_T,))
        p116_ref[...] = jnp.reshape(p1, (_T,))
        g016_ref[...] = jnp.broadcast_to(g1, (_T, 128))
        g116_ref[...] = jnp.broadcast_to(g2, (_T, 128))

        grow = (lax.broadcasted_iota(jnp.int32, (_G, _E), 0)
                .astype(jnp.float32) * _BB)
        pob = jnp.broadcast_to(po, (_G, _E))
        bexp = (jnp.sum(jnp.where(pob <= grow, 1, 0), axis=1, keepdims=True)
                - 1).astype(jnp.int32)
        prev = jnp.concatenate(
            [jnp.full((1, 1), -1, jnp.int32), bexp[:-1, :]], axis=0)
        chg = jnp.where(bexp != prev, 1, 0)
        sh3 = 1
        while sh3 < _G:
            chg = chg + jnp.concatenate(
                [jnp.zeros((sh3, 1), jnp.int32), chg[:-sh3, :]], axis=0)
            sh3 *= 2
        bexp_ref[...] = bexp
        brun_ref[...] = chg - 1
        nblk_ref[...] = (jnp.sum(pc, axis=1, keepdims=True)
                         * (1.0 / _BB)).astype(jnp.int32)


def _route(x, gate_w, gate_b, gate_out_w, gate_out_b):
    nb = 8
    bt = _T // nb
    cmap = lambda i: (0, 0)
    return pl.pallas_call(
        _route_body,
        grid=(nb,),
        in_specs=[
            pl.BlockSpec((bt, _D), lambda i: (i, 0)),
            pl.BlockSpec((_D, _H), cmap),
            pl.BlockSpec((1, _H), cmap),
            pl.BlockSpec((_H, _E), cmap),
            pl.BlockSpec((1, _E), cmap),
        ],
        out_specs=(
            pl.BlockSpec((_T,), lambda i: (0,)),
            pl.BlockSpec((_T,), lambda i: (0,)),
            pl.BlockSpec((_T, 128), cmap),
            pl.BlockSpec((_T, 128), cmap),
            pl.BlockSpec((_G, 1), cmap),
            pl.BlockSpec((_G, 1), cmap),
            pl.BlockSpec((1, 1), cmap),
        ),
        out_shape=(
            jax.ShapeDtypeStruct((_T,), jnp.int32),
            jax.ShapeDtypeStruct((_T,), jnp.int32),
            jax.ShapeDtypeStruct((_T, 128), jnp.float32),
            jax.ShapeDtypeStruct((_T, 128), jnp.float32),
            jax.ShapeDtypeStruct((_G, 1), jnp.int32),
            jax.ShapeDtypeStruct((_G, 1), jnp.int32),
            jax.ShapeDtypeStruct((1, 1), jnp.int32),
        ),
        scratch_shapes=[pltpu.VMEM((_T, _E), jnp.float32)],
        compiler_params=pltpu.CompilerParams(
            dimension_semantics=("arbitrary",)),
    )(x, gate_w, gate_b.reshape(1, _H), gate_out_w, gate_out_b.reshape(1, _E))


def _disp_body(x_hbm, p0_hbm, p1_hbm, g016_hbm, g116_hbm,
               sx_hbm, sg_hbm,
               rows_v, pos_v, gbuf_v, sems):
    wid = lax.axis_index("s") * 2 + lax.axis_index("c")
    tb = wid * _TPW
    pltpu.sync_copy(p0_hbm.at[pl.ds(tb, _TPW)], pos_v.at[0])
    pltpu.sync_copy(p1_hbm.at[pl.ds(tb, _TPW)], pos_v.at[1])
    pltpu.sync_copy(x_hbm.at[pl.ds(tb, _TPW)], rows_v)
    pltpu.sync_copy(g016_hbm.at[pl.ds(tb, _TPW)], gbuf_v.at[0])
    pltpu.sync_copy(g116_hbm.at[pl.ds(tb, _TPW)], gbuf_v.at[1])
    h0 = pltpu.async_copy(rows_v, sx_hbm.at[pos_v.at[0]], sems.at[0])
    h1 = pltpu.async_copy(rows_v, sx_hbm.at[pos_v.at[1]], sems.at[1])
    h2 = pltpu.async_copy(gbuf_v.at[0], sg_hbm.at[pos_v.at[0]], sems.at[2])
    h3 = pltpu.async_copy(gbuf_v.at[1], sg_hbm.at[pos_v.at[1]], sems.at[3])
    h0.wait()
    h1.wait()
    h2.wait()
    h3.wait()


def _dispatch_sc(x, p0, p1, g016, g116):
    mesh = plsc.VectorSubcoreMesh(core_axis_name="c", subcore_axis_name="s")
    f = functools.partial(
        pl.kernel,
        mesh=mesh,
        out_type=(
            jax.ShapeDtypeStruct((_NPAD, _D), jnp.float32),
            jax.ShapeDtypeStruct((_NPAD, 128), jnp.float32),
        ),
        scratch_types=[
            pltpu.VMEM((_TPW, _D), jnp.float32),
            pltpu.VMEM((2, _TPW), jnp.int32),
            pltpu.VMEM((2, _TPW, 128), jnp.float32),
            pltpu.SemaphoreType.DMA((4,)),
        ],
    )(_disp_body)
    return f(x, p0, p1, g016, g116)


def _mlp_body(bexp_ref, brun_ref, nblk_ref, sx_ref, sg_ref, b1_ref, b2_ref, b3_ref,
              w1_hbm, w2_hbm, w3_hbm, out_ref, w1s, w2s, w3s, sems):
    g = pl.program_id(0)
    e = bexp_ref[g, 0]
    run = brun_ref[g, 0]
    buf = run % 2

    def _start(b, ee):
        pltpu.make_async_copy(w1_hbm.at[:, ee, :], w1s.at[b], sems.at[0, b]).start()
        pltpu.make_async_copy(w2_hbm.at[:, ee, :], w2s.at[b], sems.at[1, b]).start()
        pltpu.make_async_copy(w3_hbm.at[:, ee, :], w3s.at[b], sems.at[2, b]).start()

    def _wait(b, ee):
        pltpu.make_async_copy(w1_hbm.at[:, ee, :], w1s.at[b], sems.at[0, b]).wait()
        pltpu.make_async_copy(w2_hbm.at[:, ee, :], w2s.at[b], sems.at[1, b]).wait()
        pltpu.make_async_copy(w3_hbm.at[:, ee, :], w3s.at[b], sems.at[2, b]).wait()

    @pl.when(g == 0)
    def _():
        _start(0, e)

    prev_run = brun_ref[jnp.maximum(g - 1, 0), 0]

    @pl.when((g == 0) | (run != prev_run))
    def _():
        _wait(buf, e)

    nxt = jnp.minimum(g + 1, _G - 1)
    nxt_run = brun_ref[nxt, 0]
    nxt_e = bexp_ref[nxt, 0]

    @pl.when(nxt_run != run)
    def _():
        _start(nxt_run % 2, nxt_e)

    ridx = lax.broadcasted_iota(jnp.int32, (_E, _H), 0)
    b1 = jnp.sum(jnp.where(ridx == e, b1_ref[...], 0.0), axis=0, keepdims=True)
    b2 = jnp.sum(jnp.where(ridx == e, b2_ref[...], 0.0), axis=0, keepdims=True)
    b3 = jnp.sum(jnp.where(ridx == e, b3_ref[...], 0.0), axis=0, keepdims=True)
    xb = sx_ref[...]
    h1 = jnp.maximum(
        jnp.dot(xb, w1s[buf], preferred_element_type=jnp.float32) + b1, 0.0)
    h2 = jnp.maximum(
        jnp.dot(h1, w2s[buf], preferred_element_type=jnp.float32) + b2, 0.0)
    o = jnp.dot(h2, w3s[buf], preferred_element_type=jnp.float32) + b3
    gt = sg_ref[...][:, 0:1]
    out_ref[...] = o * gt


def _grouped_mlp(bexp, brun, nblk, sx, sg, w1, b1, w2, b2, w3, b3):
    grid_spec = pltpu.PrefetchScalarGridSpec(
        num_scalar_prefetch=3,
        grid=(_G,),
        in_specs=[
            pl.BlockSpec((_BB, _D), lambda g, be, br, nbk: (g, 0)),
            pl.BlockSpec((_BB, 128), lambda g, be, br, nbk: (g, 0)),
            pl.BlockSpec((_E, _H), lambda g, be, br, nbk: (0, 0)),
            pl.BlockSpec((_E, _H), lambda g, be, br, nbk: (0, 0)),
            pl.BlockSpec((_E, _O), lambda g, be, br, nbk: (0, 0)),
            pl.BlockSpec(memory_space=pl.ANY),
            pl.BlockSpec(memory_space=pl.ANY),
            pl.BlockSpec(memory_space=pl.ANY),
        ],
        out_specs=pl.BlockSpec((_BB, _O), lambda g, be, br, nbk: (g, 0)),
        scratch_shapes=[
            pltpu.VMEM((2, _D, _H), jnp.float32),
            pltpu.VMEM((2, _H, _H), jnp.float32),
            pltpu.VMEM((2, _H, _O), jnp.float32),
            pltpu.SemaphoreType.DMA((3, 2)),
        ],
    )
    return pl.pallas_call(
        _mlp_body,
        grid_spec=grid_spec,
        out_shape=jax.ShapeDtypeStruct((_NPAD, _O), jnp.float32),
        compiler_params=pltpu.CompilerParams(
            dimension_semantics=("arbitrary",)),
    )(bexp, brun, nblk, sx, sg, b1, b2, b3, w1, w2, w3)


def _comb_body(so_hbm, p0_hbm, p1_hbm, out_hbm,
               bufa_v, bufb_v, idx_v, sems):
    wid = lax.axis_index("s") * 2 + lax.axis_index("c")
    tb = wid * _TPW
    for half in range(2):
        tc = tb + half * 32
        pltpu.sync_copy(p0_hbm.at[pl.ds(tc, 32)], idx_v.at[0])
        pltpu.sync_copy(p1_hbm.at[pl.ds(tc, 32)], idx_v.at[1])
        h0 = pltpu.async_copy(so_hbm.at[idx_v.at[0]], bufa_v, sems.at[0])
        h1 = pltpu.async_copy(so_hbm.at[idx_v.at[1]], bufb_v, sems.at[1])
        h0.wait()
        h1.wait()

        def _row_add(r, _):
            for c in range(_O // 16):
                sl = pl.ds(c * 16, 16)
                bufa_v[r, sl] = bufa_v[r, sl] + bufb_v[r, sl]
            return _

        lax.fori_loop(0, 32, _row_add, 0)
        pltpu.sync_copy(bufa_v, out_hbm.at[pl.ds(tc, 32)])


def _combine_sc(so, p016, p116):
    mesh = plsc.VectorSubcoreMesh(core_axis_name="c", subcore_axis_name="s")
    f = functools.partial(
        pl.kernel,
        mesh=mesh,
        out_type=jax.ShapeDtypeStruct((_T, _O), jnp.float32),
        scratch_types=[
            pltpu.VMEM((32, _O), jnp.float32),
            pltpu.VMEM((32, _O), jnp.float32),
            pltpu.VMEM((2, 32), jnp.int32),
            pltpu.SemaphoreType.DMA((2,)),
        ],
    )(_comb_body)
    return f(so, p016, p116)


def kernel(x, gate_w, gate_b, gate_out_w, gate_out_b,
           mlp_w1, mlp_b1, mlp_w2, mlp_b2, mlp_w3, mlp_b3):
    p0, p1, g016, g116, bexp, brun, nblk = _route(
        x, gate_w, gate_b, gate_out_w, gate_out_b)
    sx, sg = _dispatch_sc(x, p0, p1, g016, g116)
    so = _grouped_mlp(bexp, brun, nblk, sx, sg,
                      mlp_w1, mlp_b1, mlp_w2, mlp_b2, mlp_w3, mlp_b3)
    return _combine_sc(so, p0, p1)


# speedup vs baseline: 1.0493x; 1.0493x over previous
import functools

import jax
import jax.numpy as jnp
from jax import lax
from jax.experimental import pallas as pl
from jax.experimental.pallas import tpu as pltpu
from jax.experimental.pallas import tpu_sc as plsc

_T, _D, _H, _E, _O = 2048, 1024, 1024, 8, 1024
_K = 2
_BB = 384
_NPAD = 7296
_G = _NPAD // _BB
_NW = 32
_TPW = _T // _NW


def _route_body(x_ref, gw_ref, gb_ref, gow_ref, gob_ref,
                p016_ref, p116_ref, g016_ref, g116_ref,
                bexp_ref, brun_ref, nblk_ref, log_scr):
    i = pl.program_id(0)
    nb = pl.num_programs(0)
    bt = _T // nb
    h = jnp.dot(x_ref[...], gw_ref[...], preferred_element_type=jnp.float32)
    h = jnp.maximum(h + gb_ref[...], 0.0)
    log_scr[pl.ds(i * bt, bt), :] = (
        jnp.dot(h, gow_ref[...], preferred_element_type=jnp.float32)
        + gob_ref[...])

    @pl.when(i == nb - 1)
    def _():
        logits = log_scr[...]
        lane = lax.broadcasted_iota(jnp.int32, (_T, _E), 1)
        m1 = jnp.max(logits, axis=1, keepdims=True)
        i1 = jnp.min(jnp.where(logits == m1, lane, _E), axis=1, keepdims=True)
        masked = jnp.where(lane == i1, -jnp.inf, logits)
        m2 = jnp.max(masked, axis=1, keepdims=True)
        i2 = jnp.min(jnp.where(masked == m2, lane, _E), axis=1, keepdims=True)
        e2 = jnp.exp(m2 - m1)
        den = 1.0 + e2
        g1 = 1.0 / den
        g2 = e2 / den
        sel1 = jnp.where(lane == i1, 1.0, 0.0)
        sel2 = jnp.where(lane == i2, 1.0, 0.0)

        csum = sel1 + sel2
        sh = 1
        while sh < _T:
            csum = csum + jnp.concatenate(
                [jnp.zeros((sh, _E), jnp.float32), csum[:-sh, :]], axis=0)
            sh *= 2
        counts = csum[_T - 1:_T, :]
        pc = jnp.floor((counts + (_BB - 1)) * (1.0 / _BB)) * _BB
        incl = pc
        for sh2 in (1, 2, 4):
            incl = incl + jnp.concatenate(
                [jnp.zeros((1, sh2), jnp.float32), incl[:, :-sh2]], axis=1)
        po = incl - pc
        posb = csum + po - 1.0
        p0 = jnp.sum(sel1 * posb, axis=1, keepdims=True).astype(jnp.int32)
        p1 = jnp.sum(sel2 * posb, axis=1, keepdims=True).astype(jnp.int32)
        p016_ref[...] = jnp.reshape(p0, (_T,))
        p116_ref[...] = jnp.reshape(p1, (_T,))
        g016_ref[...] = jnp.broadcast_to(g1, (_T, 128))
        g116_ref[...] = jnp.broadcast_to(g2, (_T, 128))

        grow = (lax.broadcasted_iota(jnp.int32, (_G, _E), 0)
                .astype(jnp.float32) * _BB)
        pob = jnp.broadcast_to(po, (_G, _E))
        bexp = (jnp.sum(jnp.where(pob <= grow, 1, 0), axis=1, keepdims=True)
                - 1).astype(jnp.int32)
        prev = jnp.concatenate(
            [jnp.full((1, 1), -1, jnp.int32), bexp[:-1, :]], axis=0)
        chg = jnp.where(bexp != prev, 1, 0)
        sh3 = 1
        while sh3 < _G:
            chg = chg + jnp.concatenate(
                [jnp.zeros((sh3, 1), jnp.int32), chg[:-sh3, :]], axis=0)
            sh3 *= 2
        bexp_ref[...] = bexp
        brun_ref[...] = chg - 1
        nblk_ref[...] = (jnp.sum(pc, axis=1, keepdims=True)
                         * (1.0 / _BB)).astype(jnp.int32)


def _route(x, gate_w, gate_b, gate_out_w, gate_out_b):
    nb = 8
    bt = _T // nb
    cmap = lambda i: (0, 0)
    return pl.pallas_call(
        _route_body,
        grid=(nb,),
        in_specs=[
            pl.BlockSpec((bt, _D), lambda i: (i, 0)),
            pl.BlockSpec((_D, _H), cmap),
            pl.BlockSpec((1, _H), cmap),
            pl.BlockSpec((_H, _E), cmap),
            pl.BlockSpec((1, _E), cmap),
        ],
        out_specs=(
            pl.BlockSpec((_T,), lambda i: (0,)),
            pl.BlockSpec((_T,), lambda i: (0,)),
            pl.BlockSpec((_T, 128), cmap),
            pl.BlockSpec((_T, 128), cmap),
            pl.BlockSpec((_G, 1), cmap),
            pl.BlockSpec((_G, 1), cmap),
            pl.BlockSpec((1, 1), cmap),
        ),
        out_shape=(
            jax.ShapeDtypeStruct((_T,), jnp.int32),
            jax.ShapeDtypeStruct((_T,), jnp.int32),
            jax.ShapeDtypeStruct((_T, 128), jnp.float32),
            jax.ShapeDtypeStruct((_T, 128), jnp.float32),
            jax.ShapeDtypeStruct((_G, 1), jnp.int32),
            jax.ShapeDtypeStruct((_G, 1), jnp.int32),
            jax.ShapeDtypeStruct((1, 1), jnp.int32),
        ),
        scratch_shapes=[pltpu.VMEM((_T, _E), jnp.float32)],
        compiler_params=pltpu.CompilerParams(
            dimension_semantics=("arbitrary",)),
    )(x, gate_w, gate_b.reshape(1, _H), gate_out_w, gate_out_b.reshape(1, _E))


def _disp_body(x_hbm, p0_hbm, p1_hbm, g016_hbm, g116_hbm,
               sx_hbm, sg_hbm,
               rows_v, pos_v, gbuf_v, sems):
    wid = lax.axis_index("s") * 2 + lax.axis_index("c")
    tb = wid * _TPW
    pltpu.sync_copy(p0_hbm.at[pl.ds(tb, _TPW)], pos_v.at[0])
    pltpu.sync_copy(p1_hbm.at[pl.ds(tb, _TPW)], pos_v.at[1])
    pltpu.sync_copy(x_hbm.at[pl.ds(tb, _TPW)], rows_v)
    pltpu.sync_copy(g016_hbm.at[pl.ds(tb, _TPW)], gbuf_v.at[0])
    pltpu.sync_copy(g116_hbm.at[pl.ds(tb, _TPW)], gbuf_v.at[1])
    h0 = pltpu.async_copy(rows_v, sx_hbm.at[pos_v.at[0]], sems.at[0])
    h1 = pltpu.async_copy(rows_v, sx_hbm.at[pos_v.at[1]], sems.at[1])
    h2 = pltpu.async_copy(gbuf_v.at[0], sg_hbm.at[pos_v.at[0]], sems.at[2])
    h3 = pltpu.async_copy(gbuf_v.at[1], sg_hbm.at[pos_v.at[1]], sems.at[3])
    h0.wait()
    h1.wait()
    h2.wait()
    h3.wait()


def _dispatch_sc(x, p0, p1, g016, g116):
    mesh = plsc.VectorSubcoreMesh(core_axis_name="c", subcore_axis_name="s")
    f = functools.partial(
        pl.kernel,
        mesh=mesh,
        out_type=(
            jax.ShapeDtypeStruct((_NPAD, _D), jnp.float32),
            jax.ShapeDtypeStruct((_NPAD, 128), jnp.float32),
        ),
        scratch_types=[
            pltpu.VMEM((_TPW, _D), jnp.float32),
            pltpu.VMEM((2, _TPW), jnp.int32),
            pltpu.VMEM((2, _TPW, 128), jnp.float32),
            pltpu.SemaphoreType.DMA((4,)),
        ],
    )(_disp_body)
    return f(x, p0, p1, g016, g116)


def _mlp_body(bexp_ref, brun_ref, nblk_ref, sx_ref, sg_ref, b1_ref, b2_ref, b3_ref,
              w1_hbm, w2_hbm, w3_hbm, out_ref, w1s, w2s, w3s, sems):
    g = pl.program_id(0)
    e = bexp_ref[g, 0]
    run = brun_ref[g, 0]
    buf = run % 2

    def _start(b, ee):
        pltpu.make_async_copy(w1_hbm.at[:, ee, :], w1s.at[b], sems.at[0, b]).start()
        pltpu.make_async_copy(w2_hbm.at[:, ee, :], w2s.at[b], sems.at[1, b]).start()
        pltpu.make_async_copy(w3_hbm.at[:, ee, :], w3s.at[b], sems.at[2, b]).start()

    def _wait(b, ee):
        pltpu.make_async_copy(w1_hbm.at[:, ee, :], w1s.at[b], sems.at[0, b]).wait()
        pltpu.make_async_copy(w2_hbm.at[:, ee, :], w2s.at[b], sems.at[1, b]).wait()
        pltpu.make_async_copy(w3_hbm.at[:, ee, :], w3s.at[b], sems.at[2, b]).wait()

    @pl.when(g == 0)
    def _():
        _start(0, e)

    prev_run = brun_ref[jnp.maximum(g - 1, 0), 0]

    @pl.when((g == 0) | (run != prev_run))
    def _():
        _wait(buf, e)

    nxt = jnp.minimum(g + 1, _G - 1)
    nxt_run = brun_ref[nxt, 0]
    nxt_e = bexp_ref[nxt, 0]

    @pl.when(nxt_run != run)
    def _():
        _start(nxt_run % 2, nxt_e)

    @pl.when(g < nblk_ref[0, 0])
    def _():
        ridx = lax.broadcasted_iota(jnp.int32, (_E, _H), 0)
        b1 = jnp.sum(jnp.where(ridx == e, b1_ref[...], 0.0), axis=0,
                     keepdims=True)
        b2 = jnp.sum(jnp.where(ridx == e, b2_ref[...], 0.0), axis=0,
                     keepdims=True)
        b3 = jnp.sum(jnp.where(ridx == e, b3_ref[...], 0.0), axis=0,
                     keepdims=True)
        xb = sx_ref[...]
        h1 = jnp.maximum(
            jnp.dot(xb, w1s[buf], preferred_element_type=jnp.float32) + b1, 0.0)
        h2 = jnp.maximum(
            jnp.dot(h1, w2s[buf], preferred_element_type=jnp.float32) + b2, 0.0)
        o = jnp.dot(h2, w3s[buf], preferred_element_type=jnp.float32) + b3
        gt = sg_ref[...][:, 0:1]
        out_ref[...] = o * gt


def _grouped_mlp(bexp, brun, nblk, sx, sg, w1, b1, w2, b2, w3, b3):
    grid_spec = pltpu.PrefetchScalarGridSpec(
        num_scalar_prefetch=3,
        grid=(_G,),
        in_specs=[
            pl.BlockSpec((_BB, _D), lambda g, be, br, nbk: (g, 0)),
            pl.BlockSpec((_BB, 128), lambda g, be, br, nbk: (g, 0)),
            pl.BlockSpec((_E, _H), lambda g, be, br, nbk: (0, 0)),
            pl.BlockSpec((_E, _H), lambda g, be, br, nbk: (0, 0)),
            pl.BlockSpec((_E, _O), lambda g, be, br, nbk: (0, 0)),
            pl.BlockSpec(memory_space=pl.ANY),
            pl.BlockSpec(memory_space=pl.ANY),
            pl.BlockSpec(memory_space=pl.ANY),
        ],
        out_specs=pl.BlockSpec((_BB, _O), lambda g, be, br, nbk: (g, 0)),
        scratch_shapes=[
            pltpu.VMEM((2, _D, _H), jnp.float32),
            pltpu.VMEM((2, _H, _H), jnp.float32),
            pltpu.VMEM((2, _H, _O), jnp.float32),
            pltpu.SemaphoreType.DMA((3, 2)),
        ],
    )
    return pl.pallas_call(
        _mlp_body,
        grid_spec=grid_spec,
        out_shape=jax.ShapeDtypeStruct((_NPAD, _O), jnp.float32),
        compiler_params=pltpu.CompilerParams(
            dimension_semantics=("arbitrary",)),
    )(bexp, brun, nblk, sx, sg, b1, b2, b3, w1, w2, w3)


def _comb_body(so_hbm, p0_hbm, p1_hbm, out_hbm,
               bufa_v, bufb_v, idx_v, sems):
    wid = lax.axis_index("s") * 2 + lax.axis_index("c")
    tb = wid * _TPW
    for half in range(2):
        tc = tb + half * 32
        pltpu.sync_copy(p0_hbm.at[pl.ds(tc, 32)], idx_v.at[0])
        pltpu.sync_copy(p1_hbm.at[pl.ds(tc, 32)], idx_v.at[1])
        h0 = pltpu.async_copy(so_hbm.at[idx_v.at[0]], bufa_v, sems.at[0])
        h1 = pltpu.async_copy(so_hbm.at[idx_v.at[1]], bufb_v, sems.at[1])
        h0.wait()
        h1.wait()

        def _row_add(r, _):
            for c in range(_O // 16):
                sl = pl.ds(c * 16, 16)
                bufa_v[r, sl] = bufa_v[r, sl] + bufb_v[r, sl]
            return _

        lax.fori_loop(0, 32, _row_add, 0)
        pltpu.sync_copy(bufa_v, out_hbm.at[pl.ds(tc, 32)])


def _combine_sc(so, p016, p116):
    mesh = plsc.VectorSubcoreMesh(core_axis_name="c", subcore_axis_name="s")
    f = functools.partial(
        pl.kernel,
        mesh=mesh,
        out_type=jax.ShapeDtypeStruct((_T, _O), jnp.float32),
        scratch_types=[
            pltpu.VMEM((32, _O), jnp.float32),
            pltpu.VMEM((32, _O), jnp.float32),
            pltpu.VMEM((2, 32), jnp.int32),
            pltpu.SemaphoreType.DMA((2,)),
        ],
    )(_comb_body)
    return f(so, p016, p116)


def kernel(x, gate_w, gate_b, gate_out_w, gate_out_b,
           mlp_w1, mlp_b1, mlp_w2, mlp_b2, mlp_w3, mlp_b3):
    p0, p1, g016, g116, bexp, brun, nblk = _route(
        x, gate_w, gate_b, gate_out_w, gate_out_b)
    sx, sg = _dispatch_sc(x, p0, p1, g016, g116)
    so = _grouped_mlp(bexp, brun, nblk, sx, sg,
                      mlp_w1, mlp_b1, mlp_w2, mlp_b2, mlp_w3, mlp_b3)
    return _combine_sc(so, p0, p1)


# speedup vs baseline: 1.0544x; 1.0048x over previous
import functools

import jax
import jax.numpy as jnp
from jax import lax
from jax.experimental import pallas as pl
from jax.experimental.pallas import tpu as pltpu
from jax.experimental.pallas import tpu_sc as plsc

_T, _D, _H, _E, _O = 2048, 1024, 1024, 8, 1024
_K = 2
_BB = 384
_NPAD = 7296
_G = _NPAD // _BB
_NW = 32
_TPW = _T // _NW


def _route_body(x_ref, gw_ref, gb_ref, gow_ref, gob_ref,
                p016_ref, p116_ref, g016_ref, g116_ref,
                bexp_ref, brun_ref, nblk_ref, log_scr):
    i = pl.program_id(0)
    nb = pl.num_programs(0)
    bt = _T // nb
    h = jnp.dot(x_ref[...], gw_ref[...], preferred_element_type=jnp.float32)
    h = jnp.maximum(h + gb_ref[...], 0.0)
    log_scr[pl.ds(i * bt, bt), :] = (
        jnp.dot(h, gow_ref[...], preferred_element_type=jnp.float32)
        + gob_ref[...])

    @pl.when(i == nb - 1)
    def _():
        logits = log_scr[...]
        lane = lax.broadcasted_iota(jnp.int32, (_T, _E), 1)
        m1 = jnp.max(logits, axis=1, keepdims=True)
        i1 = jnp.min(jnp.where(logits == m1, lane, _E), axis=1, keepdims=True)
        masked = jnp.where(lane == i1, -jnp.inf, logits)
        m2 = jnp.max(masked, axis=1, keepdims=True)
        i2 = jnp.min(jnp.where(masked == m2, lane, _E), axis=1, keepdims=True)
        e2 = jnp.exp(m2 - m1)
        den = 1.0 + e2
        g1 = 1.0 / den
        g2 = e2 / den
        sel1 = jnp.where(lane == i1, 1.0, 0.0)
        sel2 = jnp.where(lane == i2, 1.0, 0.0)

        csum = sel1 + sel2
        sh = 1
        while sh < _T:
            csum = csum + jnp.concatenate(
                [jnp.zeros((sh, _E), jnp.float32), csum[:-sh, :]], axis=0)
            sh *= 2
        counts = csum[_T - 1:_T, :]
        pc = jnp.floor((counts + (_BB - 1)) * (1.0 / _BB)) * _BB
        incl = pc
        for sh2 in (1, 2, 4):
            incl = incl + jnp.concatenate(
                [jnp.zeros((1, sh2), jnp.float32), incl[:, :-sh2]], axis=1)
        po = incl - pc
        posb = csum + po - 1.0
        p0 = jnp.sum(sel1 * posb, axis=1, keepdims=True).astype(jnp.int32)
        p1 = jnp.sum(sel2 * posb, axis=1, keepdims=True).astype(jnp.int32)
        p016_ref[...] = jnp.reshape(p0, (_T,))
        p116_ref[...] = jnp.reshape(p1, (_T,))
        g016_ref[...] = jnp.broadcast_to(g1, (_T, 128))
        g116_ref[...] = jnp.broadcast_to(g2, (_T, 128))

        grow = (lax.broadcasted_iota(jnp.int32, (_G, _E), 0)
                .astype(jnp.float32) * _BB)
        pob = jnp.broadcast_to(po, (_G, _E))
        bexp = (jnp.sum(jnp.where(pob <= grow, 1, 0), axis=1, keepdims=True)
                - 1).astype(jnp.int32)
        prev = jnp.concatenate(
            [jnp.full((1, 1), -1, jnp.int32), bexp[:-1, :]], axis=0)
        chg = jnp.where(bexp != prev, 1, 0)
        sh3 = 1
        while sh3 < _G:
            chg = chg + jnp.concatenate(
                [jnp.zeros((sh3, 1), jnp.int32), chg[:-sh3, :]], axis=0)
            sh3 *= 2
        bexp_ref[...] = bexp
        brun_ref[...] = chg - 1
        nblk_ref[...] = (jnp.sum(pc, axis=1, keepdims=True)
                         * (1.0 / _BB)).astype(jnp.int32)


def _route(x, gate_w, gate_b, gate_out_w, gate_out_b):
    nb = 8
    bt = _T // nb
    cmap = lambda i: (0, 0)
    return pl.pallas_call(
        _route_body,
        grid=(nb,),
        in_specs=[
            pl.BlockSpec((bt, _D), lambda i: (i, 0)),
            pl.BlockSpec((_D, _H), cmap),
            pl.BlockSpec((1, _H), cmap),
            pl.BlockSpec((_H, _E), cmap),
            pl.BlockSpec((1, _E), cmap),
        ],
        out_specs=(
            pl.BlockSpec((_T,), lambda i: (0,)),
            pl.BlockSpec((_T,), lambda i: (0,)),
            pl.BlockSpec((_T, 128), cmap),
            pl.BlockSpec((_T, 128), cmap),
            pl.BlockSpec((_G, 1), cmap),
            pl.BlockSpec((_G, 1), cmap),
            pl.BlockSpec((1, 1), cmap),
        ),
        out_shape=(
            jax.ShapeDtypeStruct((_T,), jnp.int32),
            jax.ShapeDtypeStruct((_T,), jnp.int32),
            jax.ShapeDtypeStruct((_T, 128), jnp.float32),
            jax.ShapeDtypeStruct((_T, 128), jnp.float32),
            jax.ShapeDtypeStruct((_G, 1), jnp.int32),
            jax.ShapeDtypeStruct((_G, 1), jnp.int32),
            jax.ShapeDtypeStruct((1, 1), jnp.int32),
        ),
        scratch_shapes=[pltpu.VMEM((_T, _E), jnp.float32)],
        compiler_params=pltpu.CompilerParams(
            dimension_semantics=("arbitrary",)),
    )(x, gate_w, gate_b.reshape(1, _H), gate_out_w, gate_out_b.reshape(1, _E))


def _disp_body(x_hbm, p0_hbm, p1_hbm, g016_hbm, g116_hbm,
               sx_hbm, sg_hbm,
               rows_v, pos_v, gbuf_v, sems):
    wid = lax.axis_index("s") * 2 + lax.axis_index("c")
    tb = wid * _TPW
    pltpu.sync_copy(p0_hbm.at[pl.ds(tb, _TPW)], pos_v.at[0])
    pltpu.sync_copy(p1_hbm.at[pl.ds(tb, _TPW)], pos_v.at[1])
    pltpu.sync_copy(x_hbm.at[pl.ds(tb, _TPW)], rows_v)
    pltpu.sync_copy(g016_hbm.at[pl.ds(tb, _TPW)], gbuf_v.at[0])
    pltpu.sync_copy(g116_hbm.at[pl.ds(tb, _TPW)], gbuf_v.at[1])
    h0 = pltpu.async_copy(rows_v, sx_hbm.at[pos_v.at[0]], sems.at[0])
    h1 = pltpu.async_copy(rows_v, sx_hbm.at[pos_v.at[1]], sems.at[1])
    h2 = pltpu.async_copy(gbuf_v.at[0], sg_hbm.at[pos_v.at[0]], sems.at[2])
    h3 = pltpu.async_copy(gbuf_v.at[1], sg_hbm.at[pos_v.at[1]], sems.at[3])
    h0.wait()
    h1.wait()
    h2.wait()
    h3.wait()


def _dispatch_sc(x, p0, p1, g016, g116):
    mesh = plsc.VectorSubcoreMesh(core_axis_name="c", subcore_axis_name="s")
    f = functools.partial(
        pl.kernel,
        mesh=mesh,
        out_type=(
            jax.ShapeDtypeStruct((_NPAD, _D), jnp.float32),
            jax.ShapeDtypeStruct((_NPAD, 128), jnp.float32),
        ),
        scratch_types=[
            pltpu.VMEM((_TPW, _D), jnp.float32),
            pltpu.VMEM((2, _TPW), jnp.int32),
            pltpu.VMEM((2, _TPW, 128), jnp.float32),
            pltpu.SemaphoreType.DMA((4,)),
        ],
    )(_disp_body)
    return f(x, p0, p1, g016, g116)


def _mlp_body(bexp_ref, brun_ref, nblk_ref, sx_ref, sg_ref, b1_ref, b2_ref, b3_ref,
              w1_hbm, w2_hbm, w3_hbm, out_ref, w1s, w2s, w3s, sems):
    g = pl.program_id(0)
    e = bexp_ref[g, 0]
    run = brun_ref[g, 0]
    buf = run % 2

    def _start(b, ee):
        pltpu.make_async_copy(w1_hbm.at[:, ee, :], w1s.at[b], sems.at[0, b]).start()
        pltpu.make_async_copy(w2_hbm.at[:, ee, :], w2s.at[b], sems.at[1, b]).start()
        pltpu.make_async_copy(w3_hbm.at[:, ee, :], w3s.at[b], sems.at[2, b]).start()

    def _wait(b, ee):
        pltpu.make_async_copy(w1_hbm.at[:, ee, :], w1s.at[b], sems.at[0, b]).wait()
        pltpu.make_async_copy(w2_hbm.at[:, ee, :], w2s.at[b], sems.at[1, b]).wait()
        pltpu.make_async_copy(w3_hbm.at[:, ee, :], w3s.at[b], sems.at[2, b]).wait()

    @pl.when(g == 0)
    def _():
        _start(0, e)

    prev_run = brun_ref[jnp.maximum(g - 1, 0), 0]

    @pl.when((g == 0) | (run != prev_run))
    def _():
        _wait(buf, e)

    nxt = jnp.minimum(g + 1, _G - 1)
    nxt_run = brun_ref[nxt, 0]
    nxt_e = bexp_ref[nxt, 0]

    @pl.when(nxt_run != run)
    def _():
        _start(nxt_run % 2, nxt_e)

    @pl.when(g < nblk_ref[0, 0])
    def _():
        ridx = lax.broadcasted_iota(jnp.int32, (_E, _H), 0)
        b1 = jnp.sum(jnp.where(ridx == e, b1_ref[...], 0.0), axis=0,
                     keepdims=True)
        b2 = jnp.sum(jnp.where(ridx == e, b2_ref[...], 0.0), axis=0,
                     keepdims=True)
        b3 = jnp.sum(jnp.where(ridx == e, b3_ref[...], 0.0), axis=0,
                     keepdims=True)
        xb = sx_ref[...]
        h1 = jnp.maximum(
            jnp.dot(xb, w1s[buf], preferred_element_type=jnp.float32) + b1, 0.0)
        h2 = jnp.maximum(
            jnp.dot(h1, w2s[buf], preferred_element_type=jnp.float32) + b2, 0.0)
        o = jnp.dot(h2, w3s[buf], preferred_element_type=jnp.float32) + b3
        gt = sg_ref[...][:, 0:1]
        out_ref[...] = o * gt


def _grouped_mlp(bexp, brun, nblk, sx, sg, w1, b1, w2, b2, w3, b3):
    grid_spec = pltpu.PrefetchScalarGridSpec(
        num_scalar_prefetch=3,
        grid=(_G,),
        in_specs=[
            pl.BlockSpec((_BB, _D), lambda g, be, br, nbk: (g, 0)),
            pl.BlockSpec((_BB, 128), lambda g, be, br, nbk: (g, 0)),
            pl.BlockSpec((_E, _H), lambda g, be, br, nbk: (0, 0)),
            pl.BlockSpec((_E, _H), lambda g, be, br, nbk: (0, 0)),
            pl.BlockSpec((_E, _O), lambda g, be, br, nbk: (0, 0)),
            pl.BlockSpec(memory_space=pl.ANY),
            pl.BlockSpec(memory_space=pl.ANY),
            pl.BlockSpec(memory_space=pl.ANY),
        ],
        out_specs=pl.BlockSpec((_BB, _O), lambda g, be, br, nbk: (g, 0)),
        scratch_shapes=[
            pltpu.VMEM((2, _D, _H), jnp.float32),
            pltpu.VMEM((2, _H, _H), jnp.float32),
            pltpu.VMEM((2, _H, _O), jnp.float32),
            pltpu.SemaphoreType.DMA((3, 2)),
        ],
    )
    return pl.pallas_call(
        _mlp_body,
        grid_spec=grid_spec,
        out_shape=jax.ShapeDtypeStruct((_NPAD, _O), jnp.float32),
        compiler_params=pltpu.CompilerParams(
            dimension_semantics=("arbitrary",)),
    )(bexp, brun, nblk, sx, sg, b1, b2, b3, w1, w2, w3)


def _comb_body(so_hbm, p0_hbm, p1_hbm, out_hbm, a_v, b_v, idx_v, sems):
    wid = lax.axis_index("s") * 2 + lax.axis_index("c")
    tb = wid * _TPW
    pltpu.sync_copy(p0_hbm.at[pl.ds(tb, _TPW)], idx_v.at[0])
    pltpu.sync_copy(p1_hbm.at[pl.ds(tb, _TPW)], idx_v.at[1])
    handles = {}

    def _start(q):
        par = q % 2
        handles[q] = (
            pltpu.async_copy(so_hbm.at[idx_v.at[0, pl.ds(q * 16, 16)]],
                             a_v.at[par], sems.at[0, par]),
            pltpu.async_copy(so_hbm.at[idx_v.at[1, pl.ds(q * 16, 16)]],
                             b_v.at[par], sems.at[1, par]),
        )

    _start(0)
    for q in range(4):
        if q + 1 < 4:
            _start(q + 1)
        ha, hb = handles[q]
        ha.wait()
        hb.wait()
        par = q % 2

        def _row_add(r, carry, par=par):
            for c in range(_O // 16):
                sl = pl.ds(c * 16, 16)
                a_v[par, r, sl] = a_v[par, r, sl] + b_v[par, r, sl]
            return carry

        lax.fori_loop(0, 16, _row_add, 0)
        pltpu.sync_copy(a_v.at[par], out_hbm.at[pl.ds(tb + q * 16, 16)])


def _combine_sc(so, p016, p116):
    mesh = plsc.VectorSubcoreMesh(core_axis_name="c", subcore_axis_name="s")
    f = functools.partial(
        pl.kernel,
        mesh=mesh,
        out_type=jax.ShapeDtypeStruct((_T, _O), jnp.float32),
        scratch_types=[
            pltpu.VMEM((2, 16, _O), jnp.float32),
            pltpu.VMEM((2, 16, _O), jnp.float32),
            pltpu.VMEM((2, _TPW), jnp.int32),
            pltpu.SemaphoreType.DMA((2, 2)),
        ],
    )(_comb_body)
    return f(so, p016, p116)


def kernel(x, gate_w, gate_b, gate_out_w, gate_out_b,
           mlp_w1, mlp_b1, mlp_w2, mlp_b2, mlp_w3, mlp_b3):
    p0, p1, g016, g116, bexp, brun, nblk = _route(
        x, gate_w, gate_b, gate_out_w, gate_out_b)
    sx, sg = _dispatch_sc(x, p0, p1, g016, g116)
    so = _grouped_mlp(bexp, brun, nblk, sx, sg,
                      mlp_w1, mlp_b1, mlp_w2, mlp_b2, mlp_w3, mlp_b3)
    return _combine_sc(so, p0, p1)


# speedup vs baseline: 1.0686x; 1.0135x over previous
import functools

import jax
import jax.numpy as jnp
from jax import lax
from jax.experimental import pallas as pl
from jax.experimental.pallas import tpu as pltpu
from jax.experimental.pallas import tpu_sc as plsc

_T, _D, _H, _E, _O = 2048, 1024, 1024, 8, 1024
_K = 2
_BB = 384
_NPAD = 7296
_G = _NPAD // _BB
_NW = 32
_TPW = _T // _NW


def _route_body(x_ref, gw_ref, gb_ref, gow_ref, gob_ref,
                p016_ref, p116_ref, g016_ref, g116_ref,
                bexp_ref, brun_ref, nblk_ref, log_scr):
    i = pl.program_id(0)
    nb = pl.num_programs(0)
    bt = _T // nb
    h = jnp.dot(x_ref[...], gw_ref[...], preferred_element_type=jnp.float32)
    h = jnp.maximum(h + gb_ref[...], 0.0)
    log_scr[:, pl.ds(i * bt, bt)] = (
        jax.lax.dot_general(gow_ref[...], h, (((0,), (1,)), ((), ())),
                            preferred_element_type=jnp.float32)
        + gob_ref[...])

    @pl.when(i == nb - 1)
    def _():
        logits = log_scr[...]
        erow = lax.broadcasted_iota(jnp.int32, (_E, _T), 0)
        m1 = jnp.max(logits, axis=0, keepdims=True)
        i1 = jnp.min(jnp.where(logits == m1, erow, _E), axis=0, keepdims=True)
        masked = jnp.where(erow == i1, -jnp.inf, logits)
        m2 = jnp.max(masked, axis=0, keepdims=True)
        i2 = jnp.min(jnp.where(masked == m2, erow, _E), axis=0, keepdims=True)
        e2 = jnp.exp(m2 - m1)
        den = 1.0 + e2
        g1 = 1.0 / den
        g2 = e2 / den
        sel1 = jnp.where(erow == i1, 1.0, 0.0)
        sel2 = jnp.where(erow == i2, 1.0, 0.0)

        csum = sel1 + sel2
        sh = 1
        while sh < _T:
            csum = csum + jnp.concatenate(
                [jnp.zeros((_E, sh), jnp.float32), csum[:, :-sh]], axis=1)
            sh *= 2
        counts = csum[:, _T - 1:_T]
        pc = jnp.floor((counts + (_BB - 1)) * (1.0 / _BB)) * _BB
        incl = pc
        for sh2 in (1, 2, 4):
            incl = incl + jnp.concatenate(
                [jnp.zeros((sh2, 1), jnp.float32), incl[:-sh2, :]], axis=0)
        po = incl - pc
        posb = csum + po - 1.0
        p0 = jnp.sum(sel1 * posb, axis=0, keepdims=True).astype(jnp.int32)
        p1 = jnp.sum(sel2 * posb, axis=0, keepdims=True).astype(jnp.int32)
        p016_ref[...] = jnp.reshape(p0, (_T,))
        p116_ref[...] = jnp.reshape(p1, (_T,))
        g016_ref[...] = jnp.broadcast_to(jnp.reshape(g1, (_T, 1)), (_T, 128))
        g116_ref[...] = jnp.broadcast_to(jnp.reshape(g2, (_T, 1)), (_T, 128))

        gcol = (lax.broadcasted_iota(jnp.int32, (_E, _G), 1)
                .astype(jnp.float32) * _BB)
        pob = jnp.broadcast_to(po, (_E, _G))
        bexp_row = (jnp.sum(jnp.where(pob <= gcol, 1, 0), axis=0,
                            keepdims=True) - 1)
        bexp = jnp.reshape(bexp_row, (_G, 1)).astype(jnp.int32)
        prev = jnp.concatenate(
            [jnp.full((1, 1), -1, jnp.int32), bexp[:-1, :]], axis=0)
        chg = jnp.where(bexp != prev, 1, 0)
        sh3 = 1
        while sh3 < _G:
            chg = chg + jnp.concatenate(
                [jnp.zeros((sh3, 1), jnp.int32), chg[:-sh3, :]], axis=0)
            sh3 *= 2
        bexp_ref[...] = bexp
        brun_ref[...] = chg - 1
        nblk_ref[...] = jnp.sum(pc * (1.0 / _BB), axis=0,
                                keepdims=True).astype(jnp.int32)


def _route(x, gate_w, gate_b, gate_out_w, gate_out_b):
    nb = 8
    bt = _T // nb
    cmap = lambda i: (0, 0)
    return pl.pallas_call(
        _route_body,
        grid=(nb,),
        in_specs=[
            pl.BlockSpec((bt, _D), lambda i: (i, 0)),
            pl.BlockSpec((_D, _H), cmap),
            pl.BlockSpec((1, _H), cmap),
            pl.BlockSpec((_H, _E), cmap),
            pl.BlockSpec((_E, 1), cmap),
        ],
        out_specs=(
            pl.BlockSpec((_T,), lambda i: (0,)),
            pl.BlockSpec((_T,), lambda i: (0,)),
            pl.BlockSpec((_T, 128), cmap),
            pl.BlockSpec((_T, 128), cmap),
            pl.BlockSpec((_G, 1), cmap),
            pl.BlockSpec((_G, 1), cmap),
            pl.BlockSpec((1, 1), cmap),
        ),
        out_shape=(
            jax.ShapeDtypeStruct((_T,), jnp.int32),
            jax.ShapeDtypeStruct((_T,), jnp.int32),
            jax.ShapeDtypeStruct((_T, 128), jnp.float32),
            jax.ShapeDtypeStruct((_T, 128), jnp.float32),
            jax.ShapeDtypeStruct((_G, 1), jnp.int32),
            jax.ShapeDtypeStruct((_G, 1), jnp.int32),
            jax.ShapeDtypeStruct((1, 1), jnp.int32),
        ),
        scratch_shapes=[pltpu.VMEM((_E, _T), jnp.float32)],
        compiler_params=pltpu.CompilerParams(
            dimension_semantics=("arbitrary",)),
    )(x, gate_w, gate_b.reshape(1, _H), gate_out_w,
      gate_out_b.reshape(_E, 1))


def _disp_body(x_hbm, p0_hbm, p1_hbm, g016_hbm, g116_hbm,
               sx_hbm, sg_hbm,
               rows_v, pos_v, gbuf_v, sems):
    wid = lax.axis_index("s") * 2 + lax.axis_index("c")
    tb = wid * _TPW
    pltpu.sync_copy(p0_hbm.at[pl.ds(tb, _TPW)], pos_v.at[0])
    pltpu.sync_copy(p1_hbm.at[pl.ds(tb, _TPW)], pos_v.at[1])
    pltpu.sync_copy(x_hbm.at[pl.ds(tb, _TPW)], rows_v)
    pltpu.sync_copy(g016_hbm.at[pl.ds(tb, _TPW)], gbuf_v.at[0])
    pltpu.sync_copy(g116_hbm.at[pl.ds(tb, _TPW)], gbuf_v.at[1])
    h0 = pltpu.async_copy(rows_v, sx_hbm.at[pos_v.at[0]], sems.at[0])
    h1 = pltpu.async_copy(rows_v, sx_hbm.at[pos_v.at[1]], sems.at[1])
    h2 = pltpu.async_copy(gbuf_v.at[0], sg_hbm.at[pos_v.at[0]], sems.at[2])
    h3 = pltpu.async_copy(gbuf_v.at[1], sg_hbm.at[pos_v.at[1]], sems.at[3])
    h0.wait()
    h1.wait()
    h2.wait()
    h3.wait()


def _dispatch_sc(x, p0, p1, g016, g116):
    mesh = plsc.VectorSubcoreMesh(core_axis_name="c", subcore_axis_name="s")
    f = functools.partial(
        pl.kernel,
        mesh=mesh,
        out_type=(
            jax.ShapeDtypeStruct((_NPAD, _D), jnp.float32),
            jax.ShapeDtypeStruct((_NPAD, 128), jnp.float32),
        ),
        scratch_types=[
            pltpu.VMEM((_TPW, _D), jnp.float32),
            pltpu.VMEM((2, _TPW), jnp.int32),
            pltpu.VMEM((2, _TPW, 128), jnp.float32),
            pltpu.SemaphoreType.DMA((4,)),
        ],
    )(_disp_body)
    return f(x, p0, p1, g016, g116)


def _mlp_body(bexp_ref, brun_ref, nblk_ref, sx_ref, sg_ref, b1_ref, b2_ref, b3_ref,
              w1_hbm, w2_hbm, w3_hbm, out_ref, w1s, w2s, w3s, sems):
    g = pl.program_id(0)
    e = bexp_ref[g, 0]
    run = brun_ref[g, 0]
    buf = run % 2

    def _start(b, ee):
        pltpu.make_async_copy(w1_hbm.at[:, ee, :], w1s.at[b], sems.at[0, b]).start()
        pltpu.make_async_copy(w2_hbm.at[:, ee, :], w2s.at[b], sems.at[1, b]).start()
        pltpu.make_async_copy(w3_hbm.at[:, ee, :], w3s.at[b], sems.at[2, b]).start()

    def _wait(b, ee):
        pltpu.make_async_copy(w1_hbm.at[:, ee, :], w1s.at[b], sems.at[0, b]).wait()
        pltpu.make_async_copy(w2_hbm.at[:, ee, :], w2s.at[b], sems.at[1, b]).wait()
        pltpu.make_async_copy(w3_hbm.at[:, ee, :], w3s.at[b], sems.at[2, b]).wait()

    @pl.when(g == 0)
    def _():
        _start(0, e)

    prev_run = brun_ref[jnp.maximum(g - 1, 0), 0]

    @pl.when((g == 0) | (run != prev_run))
    def _():
        _wait(buf, e)

    nxt = jnp.minimum(g + 1, _G - 1)
    nxt_run = brun_ref[nxt, 0]
    nxt_e = bexp_ref[nxt, 0]

    @pl.when(nxt_run != run)
    def _():
        _start(nxt_run % 2, nxt_e)

    @pl.when(g < nblk_ref[0, 0])
    def _():
        ridx = lax.broadcasted_iota(jnp.int32, (_E, _H), 0)
        b1 = jnp.sum(jnp.where(ridx == e, b1_ref[...], 0.0), axis=0,
                     keepdims=True)
        b2 = jnp.sum(jnp.where(ridx == e, b2_ref[...], 0.0), axis=0,
                     keepdims=True)
        b3 = jnp.sum(jnp.where(ridx == e, b3_ref[...], 0.0), axis=0,
                     keepdims=True)
        xb = sx_ref[...]
        h1 = jnp.maximum(
            jnp.dot(xb, w1s[buf], preferred_element_type=jnp.float32) + b1, 0.0)
        h2 = jnp.maximum(
            jnp.dot(h1, w2s[buf], preferred_element_type=jnp.float32) + b2, 0.0)
        o = jnp.dot(h2, w3s[buf], preferred_element_type=jnp.float32) + b3
        gt = sg_ref[...][:, 0:1]
        out_ref[...] = o * gt


def _grouped_mlp(bexp, brun, nblk, sx, sg, w1, b1, w2, b2, w3, b3):
    grid_spec = pltpu.PrefetchScalarGridSpec(
        num_scalar_prefetch=3,
        grid=(_G,),
        in_specs=[
            pl.BlockSpec((_BB, _D), lambda g, be, br, nbk: (g, 0)),
            pl.BlockSpec((_BB, 128), lambda g, be, br, nbk: (g, 0)),
            pl.BlockSpec((_E, _H), lambda g, be, br, nbk: (0, 0)),
            pl.BlockSpec((_E, _H), lambda g, be, br, nbk: (0, 0)),
            pl.BlockSpec((_E, _O), lambda g, be, br, nbk: (0, 0)),
            pl.BlockSpec(memory_space=pl.ANY),
            pl.BlockSpec(memory_space=pl.ANY),
            pl.BlockSpec(memory_space=pl.ANY),
        ],
        out_specs=pl.BlockSpec((_BB, _O), lambda g, be, br, nbk: (g, 0)),
        scratch_shapes=[
            pltpu.VMEM((2, _D, _H), jnp.float32),
            pltpu.VMEM((2, _H, _H), jnp.float32),
            pltpu.VMEM((2, _H, _O), jnp.float32),
            pltpu.SemaphoreType.DMA((3, 2)),
        ],
    )
    return pl.pallas_call(
        _mlp_body,
        grid_spec=grid_spec,
        out_shape=jax.ShapeDtypeStruct((_NPAD, _O), jnp.float32),
        compiler_params=pltpu.CompilerParams(
            dimension_semantics=("arbitrary",)),
    )(bexp, brun, nblk, sx, sg, b1, b2, b3, w1, w2, w3)


def _comb_body(so_hbm, p0_hbm, p1_hbm, out_hbm, a_v, b_v, idx_v, sems):
    wid = lax.axis_index("s") * 2 + lax.axis_index("c")
    tb = wid * _TPW
    pltpu.sync_copy(p0_hbm.at[pl.ds(tb, _TPW)], idx_v.at[0])
    pltpu.sync_copy(p1_hbm.at[pl.ds(tb, _TPW)], idx_v.at[1])
    handles = {}

    def _start(q):
        par = q % 2
        handles[q] = (
            pltpu.async_copy(so_hbm.at[idx_v.at[0, pl.ds(q * 16, 16)]],
                             a_v.at[par], sems.at[0, par]),
            pltpu.async_copy(so_hbm.at[idx_v.at[1, pl.ds(q * 16, 16)]],
                             b_v.at[par], sems.at[1, par]),
        )

    _start(0)
    for q in range(4):
        if q + 1 < 4:
            _start(q + 1)
        ha, hb = handles[q]
        ha.wait()
        hb.wait()
        par = q % 2

        def _row_add(r, carry, par=par):
            for c in range(_O // 16):
                sl = pl.ds(c * 16, 16)
                a_v[par, r, sl] = a_v[par, r, sl] + b_v[par, r, sl]
            return carry

        lax.fori_loop(0, 16, _row_add, 0)
        pltpu.sync_copy(a_v.at[par], out_hbm.at[pl.ds(tb + q * 16, 16)])


def _combine_sc(so, p016, p116):
    mesh = plsc.VectorSubcoreMesh(core_axis_name="c", subcore_axis_name="s")
    f = functools.partial(
        pl.kernel,
        mesh=mesh,
        out_type=jax.ShapeDtypeStruct((_T, _O), jnp.float32),
        scratch_types=[
            pltpu.VMEM((2, 16, _O), jnp.float32),
            pltpu.VMEM((2, 16, _O), jnp.float32),
            pltpu.VMEM((2, _TPW), jnp.int32),
            pltpu.SemaphoreType.DMA((2, 2)),
        ],
    )(_comb_body)
    return f(so, p016, p116)


def kernel(x, gate_w, gate_b, gate_out_w, gate_out_b,
           mlp_w1, mlp_b1, mlp_w2, mlp_b2, mlp_w3, mlp_b3):
    p0, p1, g016, g116, bexp, brun, nblk = _route(
        x, gate_w, gate_b, gate_out_w, gate_out_b)
    sx, sg = _dispatch_sc(x, p0, p1, g016, g116)
    so = _grouped_mlp(bexp, brun, nblk, sx, sg,
                      mlp_w1, mlp_b1, mlp_w2, mlp_b2, mlp_w3, mlp_b3)
    return _combine_sc(so, p0, p1)


# speedup vs baseline: 1.0716x; 1.0028x over previous
import functools

import jax
import jax.numpy as jnp
from jax import lax
from jax.experimental import pallas as pl
from jax.experimental.pallas import tpu as pltpu
from jax.experimental.pallas import tpu_sc as plsc

_T, _D, _H, _E, _O = 2048, 1024, 1024, 8, 1024
_K = 2
_BB = 384
_NPAD = 7296
_G = _NPAD // _BB
_NW = 32
_TPW = _T // _NW


def _route_body(x_ref, gw_ref, gb_ref, gow_ref, gob_ref,
                p016_ref, p116_ref, g016_ref, g116_ref,
                bexp_ref, brun_ref, nblk_ref, log_scr):
    i = pl.program_id(0)
    nb = pl.num_programs(0)
    bt = _T // nb
    h = jnp.dot(x_ref[...], gw_ref[...], preferred_element_type=jnp.float32)
    h = jnp.maximum(h + gb_ref[...], 0.0)
    log_scr[:, pl.ds(i * bt, bt)] = (
        jax.lax.dot_general(gow_ref[...], h, (((0,), (1,)), ((), ())),
                            preferred_element_type=jnp.float32)
        + gob_ref[...])

    @pl.when(i == nb - 1)
    def _():
        logits = log_scr[...]
        erow = lax.broadcasted_iota(jnp.int32, (_E, _T), 0)
        m1 = jnp.max(logits, axis=0, keepdims=True)
        i1 = jnp.min(jnp.where(logits == m1, erow, _E), axis=0, keepdims=True)
        masked = jnp.where(erow == i1, -jnp.inf, logits)
        m2 = jnp.max(masked, axis=0, keepdims=True)
        i2 = jnp.min(jnp.where(masked == m2, erow, _E), axis=0, keepdims=True)
        e2 = jnp.exp(m2 - m1)
        den = 1.0 + e2
        g1 = 1.0 / den
        g2 = e2 / den
        sel1 = jnp.where(erow == i1, 1.0, 0.0)
        sel2 = jnp.where(erow == i2, 1.0, 0.0)

        csum = sel1 + sel2
        sh = 1
        while sh < _T:
            csum = csum + jnp.concatenate(
                [jnp.zeros((_E, sh), jnp.float32), csum[:, :-sh]], axis=1)
            sh *= 2
        counts = csum[:, _T - 1:_T]
        pc = jnp.floor((counts + (_BB - 1)) * (1.0 / _BB)) * _BB
        incl = pc
        for sh2 in (1, 2, 4):
            incl = incl + jnp.concatenate(
                [jnp.zeros((sh2, 1), jnp.float32), incl[:-sh2, :]], axis=0)
        po = incl - pc
        posb = csum + po - 1.0
        p0 = jnp.sum(sel1 * posb, axis=0, keepdims=True).astype(jnp.int32)
        p1 = jnp.sum(sel2 * posb, axis=0, keepdims=True).astype(jnp.int32)
        p016_ref[...] = jnp.reshape(p0, (_T,))
        p116_ref[...] = jnp.reshape(p1, (_T,))
        g016_ref[...] = jnp.broadcast_to(jnp.reshape(g1, (_T, 1)), (_T, 128))
        g116_ref[...] = jnp.broadcast_to(jnp.reshape(g2, (_T, 1)), (_T, 128))

        gcol = (lax.broadcasted_iota(jnp.int32, (_E, _G), 1)
                .astype(jnp.float32) * _BB)
        pob = jnp.broadcast_to(po, (_E, _G))
        bexp_row = (jnp.sum(jnp.where(pob <= gcol, 1, 0), axis=0,
                            keepdims=True) - 1)
        bexp = jnp.reshape(bexp_row, (_G, 1)).astype(jnp.int32)
        prev = jnp.concatenate(
            [jnp.full((1, 1), -1, jnp.int32), bexp[:-1, :]], axis=0)
        chg = jnp.where(bexp != prev, 1, 0)
        sh3 = 1
        while sh3 < _G:
            chg = chg + jnp.concatenate(
                [jnp.zeros((sh3, 1), jnp.int32), chg[:-sh3, :]], axis=0)
            sh3 *= 2
        bexp_ref[...] = bexp
        brun_ref[...] = chg - 1
        nblk_ref[...] = jnp.sum(pc * (1.0 / _BB), axis=0,
                                keepdims=True).astype(jnp.int32)


def _route(x, gate_w, gate_b, gate_out_w, gate_out_b):
    nb = 8
    bt = _T // nb
    cmap = lambda i: (0, 0)
    return pl.pallas_call(
        _route_body,
        grid=(nb,),
        in_specs=[
            pl.BlockSpec((bt, _D), lambda i: (i, 0)),
            pl.BlockSpec((_D, _H), cmap),
            pl.BlockSpec((1, _H), cmap),
            pl.BlockSpec((_H, _E), cmap),
            pl.BlockSpec((_E, 1), cmap),
        ],
        out_specs=(
            pl.BlockSpec((_T,), lambda i: (0,)),
            pl.BlockSpec((_T,), lambda i: (0,)),
            pl.BlockSpec((_T, 128), cmap),
            pl.BlockSpec((_T, 128), cmap),
            pl.BlockSpec((_G, 1), cmap),
            pl.BlockSpec((_G, 1), cmap),
            pl.BlockSpec((1, 1), cmap),
        ),
        out_shape=(
            jax.ShapeDtypeStruct((_T,), jnp.int32),
            jax.ShapeDtypeStruct((_T,), jnp.int32),
            jax.ShapeDtypeStruct((_T, 128), jnp.float32),
            jax.ShapeDtypeStruct((_T, 128), jnp.float32),
            jax.ShapeDtypeStruct((_G, 1), jnp.int32),
            jax.ShapeDtypeStruct((_G, 1), jnp.int32),
            jax.ShapeDtypeStruct((1, 1), jnp.int32),
        ),
        scratch_shapes=[pltpu.VMEM((_E, _T), jnp.float32)],
        compiler_params=pltpu.CompilerParams(
            dimension_semantics=("arbitrary",)),
    )(x, gate_w, gate_b.reshape(1, _H), gate_out_w,
      gate_out_b.reshape(_E, 1))


def _disp_body(x_hbm, p0_hbm, p1_hbm, g016_hbm, g116_hbm,
               sx_hbm, sg_hbm,
               rows_v, pos_v, gbuf_v, sems):
    wid = lax.axis_index("s") * 2 + lax.axis_index("c")
    tb = wid * _TPW
    pltpu.sync_copy(p0_hbm.at[pl.ds(tb, _TPW)], pos_v.at[0])
    pltpu.sync_copy(p1_hbm.at[pl.ds(tb, _TPW)], pos_v.at[1])
    pltpu.sync_copy(x_hbm.at[pl.ds(tb, _TPW)], rows_v)
    pltpu.sync_copy(g016_hbm.at[pl.ds(tb, _TPW)], gbuf_v.at[0])
    pltpu.sync_copy(g116_hbm.at[pl.ds(tb, _TPW)], gbuf_v.at[1])
    h0 = pltpu.async_copy(rows_v, sx_hbm.at[pos_v.at[0]], sems.at[0])
    h1 = pltpu.async_copy(rows_v, sx_hbm.at[pos_v.at[1]], sems.at[1])
    h2 = pltpu.async_copy(gbuf_v.at[0], sg_hbm.at[pos_v.at[0]], sems.at[2])
    h3 = pltpu.async_copy(gbuf_v.at[1], sg_hbm.at[pos_v.at[1]], sems.at[3])
    h0.wait()
    h1.wait()
    h2.wait()
    h3.wait()


def _dispatch_sc(x, p0, p1, g016, g116):
    mesh = plsc.VectorSubcoreMesh(core_axis_name="c", subcore_axis_name="s")
    f = functools.partial(
        pl.kernel,
        mesh=mesh,
        out_type=(
            jax.ShapeDtypeStruct((_NPAD, _D), jnp.float32),
            jax.ShapeDtypeStruct((_NPAD, 128), jnp.float32),
        ),
        scratch_types=[
            pltpu.VMEM((_TPW, _D), jnp.float32),
            pltpu.VMEM((2, _TPW), jnp.int32),
            pltpu.VMEM((2, _TPW, 128), jnp.float32),
            pltpu.SemaphoreType.DMA((4,)),
        ],
    )(_disp_body)
    return f(x, p0, p1, g016, g116)


def _mlp_body(bexp_ref, brun_ref, nblk_ref, sx_ref, sg_ref, b1_ref, b2_ref, b3_ref,
              w1_hbm, w2_hbm, w3_hbm, out_ref, w1s, w2s, w3s, sems):
    g = pl.program_id(0)
    e = bexp_ref[g, 0]
    run = brun_ref[g, 0]
    buf = run % 2

    def _start(b, ee):
        pltpu.make_async_copy(w1_hbm.at[:, ee, :], w1s.at[b], sems.at[0, b]).start()
        pltpu.make_async_copy(w2_hbm.at[:, ee, :], w2s.at[b], sems.at[1, b]).start()
        pltpu.make_async_copy(w3_hbm.at[:, ee, :], w3s.at[b], sems.at[2, b]).start()

    def _wait(b, ee):
        pltpu.make_async_copy(w1_hbm.at[:, ee, :], w1s.at[b], sems.at[0, b]).wait()
        pltpu.make_async_copy(w2_hbm.at[:, ee, :], w2s.at[b], sems.at[1, b]).wait()
        pltpu.make_async_copy(w3_hbm.at[:, ee, :], w3s.at[b], sems.at[2, b]).wait()

    @pl.when(g == 0)
    def _():
        _start(0, e)

    prev_run = brun_ref[jnp.maximum(g - 1, 0), 0]

    @pl.when((g == 0) | (run != prev_run))
    def _():
        _wait(buf, e)

    nxt = jnp.minimum(g + 1, _G - 1)
    nxt_run = brun_ref[nxt, 0]
    nxt_e = bexp_ref[nxt, 0]

    @pl.when(nxt_run != run)
    def _():
        _start(nxt_run % 2, nxt_e)

    @pl.when(g < nblk_ref[0, 0])
    def _():
        ridx = lax.broadcasted_iota(jnp.int32, (_E, _H), 0)
        b1 = jnp.sum(jnp.where(ridx == e, b1_ref[...], 0.0), axis=0,
                     keepdims=True)
        b2 = jnp.sum(jnp.where(ridx == e, b2_ref[...], 0.0), axis=0,
                     keepdims=True)
        b3 = jnp.sum(jnp.where(ridx == e, b3_ref[...], 0.0), axis=0,
                     keepdims=True)
        xb = sx_ref[...].astype(jnp.bfloat16)
        h1 = jnp.maximum(
            jnp.dot(xb, w1s[buf], preferred_element_type=jnp.float32) + b1,
            0.0).astype(jnp.bfloat16)
        h2 = jnp.maximum(
            jnp.dot(h1, w2s[buf], preferred_element_type=jnp.float32) + b2,
            0.0).astype(jnp.bfloat16)
        o = jnp.dot(h2, w3s[buf], preferred_element_type=jnp.float32) + b3
        gt = sg_ref[...][:, 0:1]
        out_ref[...] = o * gt


def _grouped_mlp(bexp, brun, nblk, sx, sg, w1, b1, w2, b2, w3, b3):
    grid_spec = pltpu.PrefetchScalarGridSpec(
        num_scalar_prefetch=3,
        grid=(_G,),
        in_specs=[
            pl.BlockSpec((_BB, _D), lambda g, be, br, nbk: (g, 0)),
            pl.BlockSpec((_BB, 128), lambda g, be, br, nbk: (g, 0)),
            pl.BlockSpec((_E, _H), lambda g, be, br, nbk: (0, 0)),
            pl.BlockSpec((_E, _H), lambda g, be, br, nbk: (0, 0)),
            pl.BlockSpec((_E, _O), lambda g, be, br, nbk: (0, 0)),
            pl.BlockSpec(memory_space=pl.ANY),
            pl.BlockSpec(memory_space=pl.ANY),
            pl.BlockSpec(memory_space=pl.ANY),
        ],
        out_specs=pl.BlockSpec((_BB, _O), lambda g, be, br, nbk: (g, 0)),
        scratch_shapes=[
            pltpu.VMEM((2, _D, _H), jnp.float32),
            pltpu.VMEM((2, _H, _H), jnp.float32),
            pltpu.VMEM((2, _H, _O), jnp.float32),
            pltpu.SemaphoreType.DMA((3, 2)),
        ],
    )
    return pl.pallas_call(
        _mlp_body,
        grid_spec=grid_spec,
        out_shape=jax.ShapeDtypeStruct((_NPAD, _O), jnp.float32),
        compiler_params=pltpu.CompilerParams(
            dimension_semantics=("arbitrary",)),
    )(bexp, brun, nblk, sx, sg, b1, b2, b3, w1, w2, w3)


def _comb_body(so_hbm, p0_hbm, p1_hbm, out_hbm, a_v, b_v, idx_v, sems):
    wid = lax.axis_index("s") * 2 + lax.axis_index("c")
    tb = wid * _TPW
    pltpu.sync_copy(p0_hbm.at[pl.ds(tb, _TPW)], idx_v.at[0])
    pltpu.sync_copy(p1_hbm.at[pl.ds(tb, _TPW)], idx_v.at[1])
    handles = {}

    def _start(q):
        par = q % 2
        handles[q] = (
            pltpu.async_copy(so_hbm.at[idx_v.at[0, pl.ds(q * 16, 16)]],
                             a_v.at[par], sems.at[0, par]),
            pltpu.async_copy(so_hbm.at[idx_v.at[1, pl.ds(q * 16, 16)]],
                             b_v.at[par], sems.at[1, par]),
        )

    _start(0)
    for q in range(4):
        if q + 1 < 4:
            _start(q + 1)
        ha, hb = handles[q]
        ha.wait()
        hb.wait()
        par = q % 2

        def _row_add(r, carry, par=par):
            for c in range(_O // 16):
                sl = pl.ds(c * 16, 16)
                a_v[par, r, sl] = a_v[par, r, sl] + b_v[par, r, sl]
            return carry

        lax.fori_loop(0, 16, _row_add, 0)
        pltpu.sync_copy(a_v.at[par], out_hbm.at[pl.ds(tb + q * 16, 16)])


def _combine_sc(so, p016, p116):
    mesh = plsc.VectorSubcoreMesh(core_axis_name="c", subcore_axis_name="s")
    f = functools.partial(
        pl.kernel,
        mesh=mesh,
        out_type=jax.ShapeDtypeStruct((_T, _O), jnp.float32),
        scratch_types=[
            pltpu.VMEM((2, 16, _O), jnp.float32),
            pltpu.VMEM((2, 16, _O), jnp.float32),
            pltpu.VMEM((2, _TPW), jnp.int32),
            pltpu.SemaphoreType.DMA((2, 2)),
        ],
    )(_comb_body)
    return f(so, p016, p116)


def kernel(x, gate_w, gate_b, gate_out_w, gate_out_b,
           mlp_w1, mlp_b1, mlp_w2, mlp_b2, mlp_w3, mlp_b3):
    p0, p1, g016, g116, bexp, brun, nblk = _route(
        x, gate_w, gate_b, gate_out_w, gate_out_b)
    sx, sg = _dispatch_sc(x, p0, p1, g016, g116)
    so = _grouped_mlp(bexp, brun, nblk, sx, sg,
                      mlp_w1, mlp_b1, mlp_w2, mlp_b2, mlp_w3, mlp_b3)
    return _combine_sc(so, p0, p1)


# speedup vs baseline: 1.0886x; 1.0159x over previous
import functools

import jax
import jax.numpy as jnp
from jax import lax
from jax.experimental import pallas as pl
from jax.experimental.pallas import tpu as pltpu
from jax.experimental.pallas import tpu_sc as plsc

_T, _D, _H, _E, _O = 2048, 1024, 1024, 8, 1024
_K = 2
_BB = 384
_NPAD = 7296
_G = _NPAD // _BB
_NW = 32
_TPW = _T // _NW


def _route_body(x_ref, gw_ref, gb_ref, gow_ref, gob_ref,
                p016_ref, p116_ref, g016_ref, g116_ref,
                bexp_ref, brun_ref, nblk_ref, log_scr):
    i = pl.program_id(0)
    nb = pl.num_programs(0)
    bt = _T // nb
    h = jnp.dot(x_ref[...], gw_ref[...], preferred_element_type=jnp.float32)
    h = jnp.maximum(h + gb_ref[...], 0.0)
    log_scr[:, pl.ds(i * bt, bt)] = (
        jax.lax.dot_general(gow_ref[...], h, (((0,), (1,)), ((), ())),
                            preferred_element_type=jnp.float32)
        + gob_ref[...])

    @pl.when(i == nb - 1)
    def _():
        logits = log_scr[...]
        erow = lax.broadcasted_iota(jnp.int32, (_E, _T), 0)
        m1 = jnp.max(logits, axis=0, keepdims=True)
        i1 = jnp.min(jnp.where(logits == m1, erow, _E), axis=0, keepdims=True)
        masked = jnp.where(erow == i1, -jnp.inf, logits)
        m2 = jnp.max(masked, axis=0, keepdims=True)
        i2 = jnp.min(jnp.where(masked == m2, erow, _E), axis=0, keepdims=True)
        e2 = jnp.exp(m2 - m1)
        den = 1.0 + e2
        g1 = 1.0 / den
        g2 = e2 / den
        sel1 = jnp.where(erow == i1, 1.0, 0.0)
        sel2 = jnp.where(erow == i2, 1.0, 0.0)

        csum = sel1 + sel2
        sh = 1
        while sh < _T:
            csum = csum + jnp.concatenate(
                [jnp.zeros((_E, sh), jnp.float32), csum[:, :-sh]], axis=1)
            sh *= 2
        counts = csum[:, _T - 1:_T]
        pc = jnp.floor((counts + (_BB - 1)) * (1.0 / _BB)) * _BB
        incl = pc
        for sh2 in (1, 2, 4):
            incl = incl + jnp.concatenate(
                [jnp.zeros((sh2, 1), jnp.float32), incl[:-sh2, :]], axis=0)
        po = incl - pc
        posb = csum + po - 1.0
        p0 = jnp.sum(sel1 * posb, axis=0, keepdims=True).astype(jnp.int32)
        p1 = jnp.sum(sel2 * posb, axis=0, keepdims=True).astype(jnp.int32)
        p016_ref[...] = jnp.reshape(p0, (_T,))
        p116_ref[...] = jnp.reshape(p1, (_T,))
        g016_ref[...] = jnp.broadcast_to(jnp.reshape(g1, (_T, 1)), (_T, 128))
        g116_ref[...] = jnp.broadcast_to(jnp.reshape(g2, (_T, 1)), (_T, 128))

        gcol = (lax.broadcasted_iota(jnp.int32, (_E, _G), 1)
                .astype(jnp.float32) * _BB)
        pob = jnp.broadcast_to(po, (_E, _G))
        bexp_row = (jnp.sum(jnp.where(pob <= gcol, 1, 0), axis=0,
                            keepdims=True) - 1)
        bexp = jnp.reshape(bexp_row, (_G, 1)).astype(jnp.int32)
        prev = jnp.concatenate(
            [jnp.full((1, 1), -1, jnp.int32), bexp[:-1, :]], axis=0)
        chg = jnp.where(bexp != prev, 1, 0)
        sh3 = 1
        while sh3 < _G:
            chg = chg + jnp.concatenate(
                [jnp.zeros((sh3, 1), jnp.int32), chg[:-sh3, :]], axis=0)
            sh3 *= 2
        bexp_ref[...] = bexp
        brun_ref[...] = chg - 1
        nblk_ref[...] = jnp.sum(pc * (1.0 / _BB), axis=0,
                                keepdims=True).astype(jnp.int32)


def _route(x, gate_w, gate_b, gate_out_w, gate_out_b):
    nb = 4
    bt = _T // nb
    cmap = lambda i: (0, 0)
    return pl.pallas_call(
        _route_body,
        grid=(nb,),
        in_specs=[
            pl.BlockSpec((bt, _D), lambda i: (i, 0)),
            pl.BlockSpec((_D, _H), cmap),
            pl.BlockSpec((1, _H), cmap),
            pl.BlockSpec((_H, _E), cmap),
            pl.BlockSpec((_E, 1), cmap),
        ],
        out_specs=(
            pl.BlockSpec((_T,), lambda i: (0,)),
            pl.BlockSpec((_T,), lambda i: (0,)),
            pl.BlockSpec((_T, 128), cmap),
            pl.BlockSpec((_T, 128), cmap),
            pl.BlockSpec((_G, 1), cmap),
            pl.BlockSpec((_G, 1), cmap),
            pl.BlockSpec((1, 1), cmap),
        ),
        out_shape=(
            jax.ShapeDtypeStruct((_T,), jnp.int32),
            jax.ShapeDtypeStruct((_T,), jnp.int32),
            jax.ShapeDtypeStruct((_T, 128), jnp.float32),
            jax.ShapeDtypeStruct((_T, 128), jnp.float32),
            jax.ShapeDtypeStruct((_G, 1), jnp.int32),
            jax.ShapeDtypeStruct((_G, 1), jnp.int32),
            jax.ShapeDtypeStruct((1, 1), jnp.int32),
        ),
        scratch_shapes=[pltpu.VMEM((_E, _T), jnp.float32)],
        compiler_params=pltpu.CompilerParams(
            dimension_semantics=("arbitrary",)),
    )(x, gate_w, gate_b.reshape(1, _H), gate_out_w,
      gate_out_b.reshape(_E, 1))


def _disp_body(x_hbm, p0_hbm, p1_hbm, g016_hbm, g116_hbm,
               sx_hbm, sg_hbm,
               rows_v, pos_v, gbuf_v, sems):
    wid = lax.axis_index("s") * 2 + lax.axis_index("c")
    tb = wid * _TPW
    pltpu.sync_copy(p0_hbm.at[pl.ds(tb, _TPW)], pos_v.at[0])
    pltpu.sync_copy(p1_hbm.at[pl.ds(tb, _TPW)], pos_v.at[1])
    pltpu.sync_copy(x_hbm.at[pl.ds(tb, _TPW)], rows_v)
    pltpu.sync_copy(g016_hbm.at[pl.ds(tb, _TPW)], gbuf_v.at[0])
    pltpu.sync_copy(g116_hbm.at[pl.ds(tb, _TPW)], gbuf_v.at[1])
    h0 = pltpu.async_copy(rows_v, sx_hbm.at[pos_v.at[0]], sems.at[0])
    h1 = pltpu.async_copy(rows_v, sx_hbm.at[pos_v.at[1]], sems.at[1])
    h2 = pltpu.async_copy(gbuf_v.at[0], sg_hbm.at[pos_v.at[0]], sems.at[2])
    h3 = pltpu.async_copy(gbuf_v.at[1], sg_hbm.at[pos_v.at[1]], sems.at[3])
    h0.wait()
    h1.wait()
    h2.wait()
    h3.wait()


def _dispatch_sc(x, p0, p1, g016, g116):
    mesh = plsc.VectorSubcoreMesh(core_axis_name="c", subcore_axis_name="s")
    f = functools.partial(
        pl.kernel,
        mesh=mesh,
        out_type=(
            jax.ShapeDtypeStruct((_NPAD, _D), jnp.float32),
            jax.ShapeDtypeStruct((_NPAD, 128), jnp.float32),
        ),
        scratch_types=[
            pltpu.VMEM((_TPW, _D), jnp.float32),
            pltpu.VMEM((2, _TPW), jnp.int32),
            pltpu.VMEM((2, _TPW, 128), jnp.float32),
            pltpu.SemaphoreType.DMA((4,)),
        ],
    )(_disp_body)
    return f(x, p0, p1, g016, g116)


def _mlp_body(bexp_ref, brun_ref, nblk_ref, sx_ref, sg_ref, b1_ref, b2_ref, b3_ref,
              w1_hbm, w2_hbm, w3_hbm, out_ref, w1s, w2s, w3s, sems):
    g = pl.program_id(0)
    e = bexp_ref[g, 0]
    run = brun_ref[g, 0]
    buf = run % 2

    def _start(b, ee):
        pltpu.make_async_copy(w1_hbm.at[:, ee, :], w1s.at[b], sems.at[0, b]).start()
        pltpu.make_async_copy(w2_hbm.at[:, ee, :], w2s.at[b], sems.at[1, b]).start()
        pltpu.make_async_copy(w3_hbm.at[:, ee, :], w3s.at[b], sems.at[2, b]).start()

    def _wait(b, ee):
        pltpu.make_async_copy(w1_hbm.at[:, ee, :], w1s.at[b], sems.at[0, b]).wait()
        pltpu.make_async_copy(w2_hbm.at[:, ee, :], w2s.at[b], sems.at[1, b]).wait()
        pltpu.make_async_copy(w3_hbm.at[:, ee, :], w3s.at[b], sems.at[2, b]).wait()

    @pl.when(g == 0)
    def _():
        _start(0, e)

    prev_run = brun_ref[jnp.maximum(g - 1, 0), 0]

    @pl.when((g == 0) | (run != prev_run))
    def _():
        _wait(buf, e)

    nxt = jnp.minimum(g + 1, _G - 1)
    nxt_run = brun_ref[nxt, 0]
    nxt_e = bexp_ref[nxt, 0]

    @pl.when(nxt_run != run)
    def _():
        _start(nxt_run % 2, nxt_e)

    @pl.when(g < nblk_ref[0, 0])
    def _():
        ridx = lax.broadcasted_iota(jnp.int32, (_E, _H), 0)
        b1 = jnp.sum(jnp.where(ridx == e, b1_ref[...], 0.0), axis=0,
                     keepdims=True)
        b2 = jnp.sum(jnp.where(ridx == e, b2_ref[...], 0.0), axis=0,
                     keepdims=True)
        b3 = jnp.sum(jnp.where(ridx == e, b3_ref[...], 0.0), axis=0,
                     keepdims=True)
        xb = sx_ref[...].astype(jnp.bfloat16)
        h1 = jnp.maximum(
            jnp.dot(xb, w1s[buf], preferred_element_type=jnp.float32) + b1,
            0.0).astype(jnp.bfloat16)
        h2 = jnp.maximum(
            jnp.dot(h1, w2s[buf], preferred_element_type=jnp.float32) + b2,
            0.0).astype(jnp.bfloat16)
        o = jnp.dot(h2, w3s[buf], preferred_element_type=jnp.float32) + b3
        gt = sg_ref[...][:, 0:1]
        out_ref[...] = o * gt


def _grouped_mlp(bexp, brun, nblk, sx, sg, w1, b1, w2, b2, w3, b3):
    grid_spec = pltpu.PrefetchScalarGridSpec(
        num_scalar_prefetch=3,
        grid=(_G,),
        in_specs=[
            pl.BlockSpec((_BB, _D), lambda g, be, br, nbk: (g, 0)),
            pl.BlockSpec((_BB, 128), lambda g, be, br, nbk: (g, 0)),
            pl.BlockSpec((_E, _H), lambda g, be, br, nbk: (0, 0)),
            pl.BlockSpec((_E, _H), lambda g, be, br, nbk: (0, 0)),
            pl.BlockSpec((_E, _O), lambda g, be, br, nbk: (0, 0)),
            pl.BlockSpec(memory_space=pl.ANY),
            pl.BlockSpec(memory_space=pl.ANY),
            pl.BlockSpec(memory_space=pl.ANY),
        ],
        out_specs=pl.BlockSpec((_BB, _O), lambda g, be, br, nbk: (g, 0)),
        scratch_shapes=[
            pltpu.VMEM((2, _D, _H), jnp.float32),
            pltpu.VMEM((2, _H, _H), jnp.float32),
            pltpu.VMEM((2, _H, _O), jnp.float32),
            pltpu.SemaphoreType.DMA((3, 2)),
        ],
    )
    return pl.pallas_call(
        _mlp_body,
        grid_spec=grid_spec,
        out_shape=jax.ShapeDtypeStruct((_NPAD, _O), jnp.float32),
        compiler_params=pltpu.CompilerParams(
            dimension_semantics=("arbitrary",)),
    )(bexp, brun, nblk, sx, sg, b1, b2, b3, w1, w2, w3)


def _comb_body(so_hbm, p0_hbm, p1_hbm, out_hbm, a_v, b_v, idx_v, sems):
    wid = lax.axis_index("s") * 2 + lax.axis_index("c")
    tb = wid * _TPW
    pltpu.sync_copy(p0_hbm.at[pl.ds(tb, _TPW)], idx_v.at[0])
    pltpu.sync_copy(p1_hbm.at[pl.ds(tb, _TPW)], idx_v.at[1])
    handles = {}

    def _start(q):
        par = q % 2
        handles[q] = (
            pltpu.async_copy(so_hbm.at[idx_v.at[0, pl.ds(q * 16, 16)]],
                             a_v.at[par], sems.at[0, par]),
            pltpu.async_copy(so_hbm.at[idx_v.at[1, pl.ds(q * 16, 16)]],
                             b_v.at[par], sems.at[1, par]),
        )

    _start(0)
    for q in range(4):
        if q + 1 < 4:
            _start(q + 1)
        ha, hb = handles[q]
        ha.wait()
        hb.wait()
        par = q % 2

        def _row_add(r, carry, par=par):
            for c in range(_O // 16):
                sl = pl.ds(c * 16, 16)
                a_v[par, r, sl] = a_v[par, r, sl] + b_v[par, r, sl]
            return carry

        lax.fori_loop(0, 16, _row_add, 0)
        pltpu.sync_copy(a_v.at[par], out_hbm.at[pl.ds(tb + q * 16, 16)])


def _combine_sc(so, p016, p116):
    mesh = plsc.VectorSubcoreMesh(core_axis_name="c", subcore_axis_name="s")
    f = functools.partial(
        pl.kernel,
        mesh=mesh,
        out_type=jax.ShapeDtypeStruct((_T, _O), jnp.float32),
        scratch_types=[
            pltpu.VMEM((2, 16, _O), jnp.float32),
            pltpu.VMEM((2, 16, _O), jnp.float32),
            pltpu.VMEM((2, _TPW), jnp.int32),
            pltpu.SemaphoreType.DMA((2, 2)),
        ],
    )(_comb_body)
    return f(so, p016, p116)


def kernel(x, gate_w, gate_b, gate_out_w, gate_out_b,
           mlp_w1, mlp_b1, mlp_w2, mlp_b2, mlp_w3, mlp_b3):
    p0, p1, g016, g116, bexp, brun, nblk = _route(
        x, gate_w, gate_b, gate_out_w, gate_out_b)
    sx, sg = _dispatch_sc(x, p0, p1, g016, g116)
    so = _grouped_mlp(bexp, brun, nblk, sx, sg,
                      mlp_w1, mlp_b1, mlp_w2, mlp_b2, mlp_w3, mlp_b3)
    return _combine_sc(so, p0, p1)
